# Initial kernel scaffold; baseline (speedup 1.0000x reference)
import jax
import jax.numpy as jnp
from jax.experimental import pallas as pl


def kernel(log_scale, unscaled_x, idx, total_slots):
    raise NotImplementedError("write your pallas kernel here")


# speedup vs baseline: 53.8930x; 53.8930x over previous
import functools

import jax
import jax.numpy as jnp
from jax import lax
from jax.experimental import pallas as pl
from jax.experimental.pallas import tpu as pltpu
from jax.experimental.pallas import tpu_sc as plsc

SHAPE = (8192, 2048)
TOTAL = SHAPE[0] * SHAPE[1]
L = 16
C = 16384
W = C + 2 * L
NBLK = TOTAL // C
NWORK = 32
CPW = NBLK // NWORK
SB = NBLK + L


def _sc_scatter(params, starts, tail_idx, tail_val, idx_p, val_p, npad):
    clamp = max(0, (npad - W)) & ~7

    mesh = plsc.VectorSubcoreMesh(core_axis_name="c", subcore_axis_name="s",
                                  num_cores=2, num_subcores=16)

    @functools.partial(
        pl.kernel,
        out_type=jax.ShapeDtypeStruct((TOTAL,), jnp.float32),
        mesh=mesh,
        compiler_params=pltpu.CompilerParams(needs_layout_passes=False),
        scratch_types=[
            pltpu.VMEM((2 * L,), jnp.float32),
            pltpu.VMEM((SB,), jnp.int32),
            pltpu.VMEM((L,), jnp.int32),
            pltpu.VMEM((L,), jnp.float32),
            pltpu.VMEM((W,), jnp.int32),
            pltpu.VMEM((W,), jnp.float32),
            pltpu.VMEM((C,), jnp.float32),
        ],
    )
    def k(params_h, starts_h, tidx_h, tval_h, idx_h, val_h, out_h,
          params_v, starts_v, tidx_v, tval_v, idxw_v, valw_v, buf_v):
        wid = lax.axis_index("s") * 2 + lax.axis_index("c")
        pltpu.sync_copy(params_h, params_v)
        pltpu.sync_copy(starts_h, starts_v)
        pltpu.sync_copy(tidx_h, tidx_v)
        pltpu.sync_copy(tval_h, tval_v)
        scale_vec = jnp.exp(params_v[pl.ds(0, L)])
        zvec = params_v[pl.ds(L, L)]
        tiv = tidx_v[...]
        tvv = tval_v[...] * scale_vec

        def chunk_body(kk, carry):
            blk = wid * CPW + kk
            lo = blk * C
            sv = starts_v[pl.ds(blk, L)]
            a = sv[0]
            b = sv[1]
            a8 = pl.multiple_of(jnp.minimum(a & ~7, clamp), 8)

            def zbody(j, c):
                buf_v[pl.ds(j * L, L)] = zvec
                return c
            lax.fori_loop(0, C // L, zbody, 0, unroll=4)

            pltpu.sync_copy(idx_h.at[pl.ds(a8, W)], idxw_v)
            pltpu.sync_copy(val_h.at[pl.ds(a8, W)], valw_v)

            ng = (b - a8 + (L - 1)) // L
            lov = jnp.full((L,), lo, jnp.int32)

            def sbody(j, c):
                iv = idxw_v[pl.ds(j * L, L)]
                vv = valw_v[pl.ds(j * L, L)]
                rel = iv - lov
                m = (rel >= 0) & (rel < C)
                relc = jnp.clip(rel, 0, C - 1)
                plsc.store_scatter(buf_v, [relc], vv * scale_vec, mask=m)
                return c
            lax.fori_loop(0, ng, sbody, 0)

            trel = tiv - lov
            tm = (trel >= 0) & (trel < C)
            trelc = jnp.clip(trel, 0, C - 1)
            plsc.store_scatter(buf_v, [trelc], tvv, mask=tm)

            pltpu.sync_copy(buf_v, out_h.at[pl.ds(lo, C)])
            return carry

        lax.fori_loop(0, CPW, chunk_body, 0)

    return k(params, starts, tail_idx, tail_val, idx_p, val_p)


def kernel(log_scale, unscaled_x, idx, total_slots):
    n = idx.shape[0]
    zofs = (jnp.asarray(total_slots, jnp.float32) - jnp.float32(TOTAL))
    params = jnp.concatenate(
        [jnp.full((L,), jnp.asarray(log_scale, jnp.float32)),
         jnp.full((L,), zofs)])

    idx = idx.astype(jnp.int32)
    vals = unscaled_x.astype(jnp.float32)
    if n < W:
        pad = W - n
        idx_p = jnp.concatenate([idx, jnp.full((pad,), TOTAL, jnp.int32)])
        val_p = jnp.concatenate([vals, jnp.zeros((pad,), jnp.float32)])
        npad = W
    else:
        idx_p, val_p, npad = idx, vals, n

    bounds = jnp.arange(0, TOTAL + 1, C, dtype=jnp.int32)
    starts = jnp.searchsorted(idx_p, bounds, side="left").astype(jnp.int32)
    starts = jnp.concatenate(
        [starts, jnp.full((SB - NBLK - 1,), jnp.int32(npad))])

    t = max(0, n - L)
    tail_idx = lax.dynamic_slice_in_dim(idx_p, min(t, npad - L), L)
    tail_val = lax.dynamic_slice_in_dim(val_p, min(t, npad - L), L)

    out = _sc_scatter(params, starts, tail_idx, tail_val, idx_p, val_p, npad)
    return out.reshape(SHAPE)


# speedup vs baseline: 62.2756x; 1.1555x over previous
import functools

import jax
import jax.numpy as jnp
from jax import lax
from jax.experimental import pallas as pl
from jax.experimental.pallas import tpu as pltpu
from jax.experimental.pallas import tpu_sc as plsc

SHAPE = (8192, 2048)
TOTAL = SHAPE[0] * SHAPE[1]
L = 16
C = 16384
W = C + 2 * L
NBLK = TOTAL // C
NWORK = 32
CPW = NBLK // NWORK
SB = NBLK + 2 * L


def _sc_scatter(params, starts, tail_idx, tail_val, idx_p, val_p, npad):
    clamp = max(0, (npad - W)) & ~7

    mesh = plsc.VectorSubcoreMesh(core_axis_name="c", subcore_axis_name="s",
                                  num_cores=2, num_subcores=16)

    @functools.partial(
        pl.kernel,
        out_type=jax.ShapeDtypeStruct((TOTAL,), jnp.float32),
        mesh=mesh,
        compiler_params=pltpu.CompilerParams(needs_layout_passes=False),
        scratch_types=[
            pltpu.VMEM((2 * L,), jnp.float32),
            pltpu.VMEM((SB,), jnp.int32),
            pltpu.VMEM((L,), jnp.int32),
            pltpu.VMEM((L,), jnp.float32),
            pltpu.VMEM((W,), jnp.int32),
            pltpu.VMEM((W,), jnp.int32),
            pltpu.VMEM((W,), jnp.float32),
            pltpu.VMEM((W,), jnp.float32),
            pltpu.VMEM((C,), jnp.float32),
            pltpu.VMEM((C,), jnp.float32),
            pltpu.SemaphoreType.DMA,
            pltpu.SemaphoreType.DMA,
            pltpu.SemaphoreType.DMA,
            pltpu.SemaphoreType.DMA,
        ],
    )
    def k(params_h, starts_h, tidx_h, tval_h, idx_h, val_h, out_h,
          params_v, starts_v, tidx_v, tval_v,
          idxw0, idxw1, valw0, valw1, buf0, buf1,
          wsem0, wsem1, osem0, osem1):
        idxw = (idxw0, idxw1)
        valw = (valw0, valw1)
        buf = (buf0, buf1)
        wsem = (wsem0, wsem1)
        osem = (osem0, osem1)

        wid = lax.axis_index("s") * 2 + lax.axis_index("c")
        pltpu.sync_copy(params_h, params_v)
        pltpu.sync_copy(starts_h, starts_v)
        pltpu.sync_copy(tidx_h, tidx_v)
        pltpu.sync_copy(tval_h, tval_v)
        scale_vec = jnp.exp(params_v[pl.ds(0, L)])
        zvec = params_v[pl.ds(L, L)]
        tiv = tidx_v[...]
        tvv = tval_v[...] * scale_vec
        cu = jnp.full((L,), C, jnp.uint32)

        def win_a8(k_):
            blk = wid * CPW + k_
            sv = starts_v[pl.ds(blk, L)]
            a8 = pl.multiple_of(jnp.minimum(sv[0] & ~7, clamp), 8)
            return blk, sv, a8

        def start_win(k_, p):
            _, _, a8 = win_a8(k_)
            pltpu.async_copy(idx_h.at[pl.ds(a8, W)], idxw[p], wsem[p])
            pltpu.async_copy(val_h.at[pl.ds(a8, W)], valw[p], wsem[p])

        start_win(0, 0)

        def body(kk2, carry):
            for p in (0, 1):
                k_ = 2 * kk2 + p
                blk, sv, a8 = win_a8(k_)
                lo = blk * C
                b = sv[1]
                start_win(k_ + 1, 1 - p)
                pltpu.make_async_copy(
                    idx_h.at[pl.ds(a8, W)], idxw[p], wsem[p]).wait()
                pltpu.make_async_copy(
                    val_h.at[pl.ds(a8, W)], valw[p], wsem[p]).wait()

                @pl.when(kk2 > 0)
                def _():
                    pltpu.make_async_copy(
                        buf[p], out_h.at[pl.ds(0, C)], osem[p]).wait()

                def zbody(j, c):
                    buf[p][pl.ds(j * L, L)] = zvec
                    return c
                lax.fori_loop(0, C // L, zbody, 0, unroll=8)

                ng = (b - a8 + (L - 1)) // L
                lov = jnp.full((L,), lo, jnp.int32)

                def sbody(j, c):
                    iv = idxw[p][pl.ds(j * L, L)]
                    vv = valw[p][pl.ds(j * L, L)]
                    rel = iv - lov
                    m = plsc.bitcast(rel, jnp.uint32) < cu
                    plsc.store_scatter(buf[p], [rel], vv * scale_vec, mask=m)
                    return c
                lax.fori_loop(0, ng, sbody, 0)

                trel = tiv - lov
                tm = plsc.bitcast(trel, jnp.uint32) < cu
                plsc.store_scatter(buf[p], [trel], tvv, mask=tm)

                pltpu.async_copy(buf[p], out_h.at[pl.ds(lo, C)], osem[p])
            return carry

        lax.fori_loop(0, CPW // 2, body, 0)

        pltpu.make_async_copy(idx_h.at[pl.ds(0, W)], idxw[0], wsem[0]).wait()
        pltpu.make_async_copy(val_h.at[pl.ds(0, W)], valw[0], wsem[0]).wait()
        pltpu.make_async_copy(buf[0], out_h.at[pl.ds(0, C)], osem[0]).wait()
        pltpu.make_async_copy(buf[1], out_h.at[pl.ds(0, C)], osem[1]).wait()

    return k(params, starts, tail_idx, tail_val, idx_p, val_p)


def kernel(log_scale, unscaled_x, idx, total_slots):
    n = idx.shape[0]
    zofs = (jnp.asarray(total_slots, jnp.float32) - jnp.float32(TOTAL))
    params = jnp.concatenate(
        [jnp.full((L,), jnp.asarray(log_scale, jnp.float32)),
         jnp.full((L,), zofs)])

    idx = idx.astype(jnp.int32)
    vals = unscaled_x.astype(jnp.float32)
    if n < W:
        pad = W - n
        idx_p = jnp.concatenate([idx, jnp.full((pad,), TOTAL, jnp.int32)])
        val_p = jnp.concatenate([vals, jnp.zeros((pad,), jnp.float32)])
        npad = W
    else:
        idx_p, val_p, npad = idx, vals, n

    bounds = jnp.arange(0, TOTAL + 1, C, dtype=jnp.int32)
    starts = jnp.searchsorted(idx_p, bounds, side="left").astype(jnp.int32)
    starts = jnp.concatenate(
        [starts, jnp.full((SB - NBLK - 1,), jnp.int32(npad))])

    t = max(0, n - L)
    tail_idx = lax.dynamic_slice_in_dim(idx_p, min(t, npad - L), L)
    tail_val = lax.dynamic_slice_in_dim(val_p, min(t, npad - L), L)

    out = _sc_scatter(params, starts, tail_idx, tail_val, idx_p, val_p, npad)
    return out.reshape(SHAPE)


# speedup vs baseline: 183.8383x; 2.9520x over previous
import functools

import jax
import jax.numpy as jnp
from jax import lax
from jax.experimental import pallas as pl
from jax.experimental.pallas import tpu as pltpu
from jax.experimental.pallas import tpu_sc as plsc

SHAPE = (8192, 2048)
TOTAL = SHAPE[0] * SHAPE[1]
L = 16
C = 16384
W = C + 2 * L
NBLK = TOTAL // C
NWORK = 32
CPW = NBLK // NWORK
NBG = (CPW + 1 + L - 1) // L + 1


def _sc_scatter(params, tail_idx, tail_val, idx_p, val_p, npad):
    clamp = max(0, (npad - W)) & ~7
    steps = int(npad).bit_length() + 1

    mesh = plsc.VectorSubcoreMesh(core_axis_name="c", subcore_axis_name="s",
                                  num_cores=2, num_subcores=16)

    @functools.partial(
        pl.kernel,
        out_type=jax.ShapeDtypeStruct((TOTAL,), jnp.float32),
        mesh=mesh,
        compiler_params=pltpu.CompilerParams(needs_layout_passes=False),
        scratch_types=[
            pltpu.VMEM((2 * L,), jnp.float32),
            pltpu.VMEM((NBG * L,), jnp.int32),
            pltpu.VMEM((NBG * L,), jnp.int32),
            pltpu.VMEM((NBG * L,), jnp.int32),
            pltpu.VMEM((L,), jnp.int32),
            pltpu.VMEM((L,), jnp.float32),
            pltpu.VMEM((W,), jnp.int32),
            pltpu.VMEM((W,), jnp.int32),
            pltpu.VMEM((W,), jnp.float32),
            pltpu.VMEM((W,), jnp.float32),
            pltpu.VMEM((C,), jnp.float32),
            pltpu.VMEM((C,), jnp.float32),
            pltpu.SemaphoreType.DMA,
            pltpu.SemaphoreType.DMA,
            pltpu.SemaphoreType.DMA,
            pltpu.SemaphoreType.DMA,
            pltpu.SemaphoreType.DMA,
        ],
    )
    def k(params_h, tidx_h, tval_h, idx_h, val_h, out_h,
          params_v, starts_v, probe_v, pdst_v, tidx_v, tval_v,
          idxw0, idxw1, valw0, valw1, buf0, buf1,
          psem, wsem0, wsem1, osem0, osem1):
        idxw = (idxw0, idxw1)
        valw = (valw0, valw1)
        buf = (buf0, buf1)
        wsem = (wsem0, wsem1)
        osem = (osem0, osem1)

        wid = lax.axis_index("s") * 2 + lax.axis_index("c")
        pltpu.sync_copy(params_h, params_v)
        pltpu.sync_copy(tidx_h, tidx_v)
        pltpu.sync_copy(tval_h, tval_v)
        scale_vec = jnp.exp(params_v[pl.ds(0, L)])
        zvec = params_v[pl.ds(L, L)]
        tiv = tidx_v[...]
        tvv = tval_v[...] * scale_vec
        cu = jnp.full((L,), C, jnp.uint32)

        lane = lax.iota(jnp.int32, L)
        base_b = wid * CPW
        bval = []
        for g in range(NBG):
            bval.append((base_b + g * L + lane) * C)
        los = [jnp.zeros((L,), jnp.int32)] * NBG
        his = [jnp.full((L,), npad, jnp.int32)] * NBG

        def search_body(_, carry):
            los, his = carry
            mids = []
            for g in range(NBG):
                mid = (los[g] + his[g]) >> 1
                mids.append(mid)
                probe_v[pl.ds(g * L, L)] = jnp.minimum(
                    mid, jnp.int32(npad - 1))
            cp = pltpu.async_copy(
                idx_h.at[probe_v], pdst_v, psem)
            cp.wait()
            nlos, nhis = [], []
            for g in range(NBG):
                gv = pdst_v[pl.ds(g * L, L)]
                active = los[g] < his[g]
                less = gv < bval[g]
                nlos.append(jnp.where(active & less, mids[g] + 1, los[g]))
                nhis.append(jnp.where(active & (~less), mids[g], his[g]))
            return nlos, nhis

        los, his = lax.fori_loop(0, steps, search_body, (los, his))
        for g in range(NBG):
            starts_v[pl.ds(g * L, L)] = his[g]

        def win_a8(k_):
            sv = starts_v[pl.ds(k_, L)]
            a8 = pl.multiple_of(jnp.minimum(sv[0] & ~7, clamp), 8)
            return sv, a8

        def start_win(k_, p):
            _, a8 = win_a8(k_)
            pltpu.async_copy(idx_h.at[pl.ds(a8, W)], idxw[p], wsem[p])
            pltpu.async_copy(val_h.at[pl.ds(a8, W)], valw[p], wsem[p])

        start_win(0, 0)

        def body(kk2, carry):
            for p in (0, 1):
                k_ = 2 * kk2 + p
                sv, a8 = win_a8(k_)
                lo = (wid * CPW + k_) * C
                b = sv[1]
                start_win(k_ + 1, 1 - p)
                pltpu.make_async_copy(
                    idx_h.at[pl.ds(a8, W)], idxw[p], wsem[p]).wait()
                pltpu.make_async_copy(
                    val_h.at[pl.ds(a8, W)], valw[p], wsem[p]).wait()

                @pl.when(kk2 > 0)
                def _():
                    pltpu.make_async_copy(
                        buf[p], out_h.at[pl.ds(0, C)], osem[p]).wait()

                def zbody(j, c):
                    buf[p][pl.ds(j * L, L)] = zvec
                    return c
                lax.fori_loop(0, C // L, zbody, 0, unroll=8)

                ng = (b - a8 + (L - 1)) // L
                lov = jnp.full((L,), lo, jnp.int32)

                def sbody(j, c):
                    iv = idxw[p][pl.ds(j * L, L)]
                    vv = valw[p][pl.ds(j * L, L)]
                    rel = iv - lov
                    m = plsc.bitcast(rel, jnp.uint32) < cu
                    plsc.store_scatter(buf[p], [rel], vv * scale_vec, mask=m)
                    return c
                lax.fori_loop(0, ng, sbody, 0)

                trel = tiv - lov
                tm = plsc.bitcast(trel, jnp.uint32) < cu
                plsc.store_scatter(buf[p], [trel], tvv, mask=tm)

                pltpu.async_copy(buf[p], out_h.at[pl.ds(lo, C)], osem[p])
            return carry

        lax.fori_loop(0, CPW // 2, body, 0)

        pltpu.make_async_copy(idx_h.at[pl.ds(0, W)], idxw[0], wsem[0]).wait()
        pltpu.make_async_copy(val_h.at[pl.ds(0, W)], valw[0], wsem[0]).wait()
        pltpu.make_async_copy(buf[0], out_h.at[pl.ds(0, C)], osem[0]).wait()
        pltpu.make_async_copy(buf[1], out_h.at[pl.ds(0, C)], osem[1]).wait()

    return k(params, tail_idx, tail_val, idx_p, val_p)


def kernel(log_scale, unscaled_x, idx, total_slots):
    n = idx.shape[0]
    zofs = (jnp.asarray(total_slots, jnp.float32) - jnp.float32(TOTAL))
    params = jnp.concatenate(
        [jnp.full((L,), jnp.asarray(log_scale, jnp.float32)),
         jnp.full((L,), zofs)])

    idx = idx.astype(jnp.int32)
    vals = unscaled_x.astype(jnp.float32)
    if n < W:
        pad = W - n
        idx_p = jnp.concatenate([idx, jnp.full((pad,), TOTAL, jnp.int32)])
        val_p = jnp.concatenate([vals, jnp.zeros((pad,), jnp.float32)])
        npad = W
    else:
        idx_p, val_p, npad = idx, vals, n

    t = max(0, n - L)
    tail_idx = lax.dynamic_slice_in_dim(idx_p, min(t, npad - L), L)
    tail_val = lax.dynamic_slice_in_dim(val_p, min(t, npad - L), L)

    out = _sc_scatter(params, tail_idx, tail_val, idx_p, val_p, npad)
    return out.reshape(SHAPE)


# speedup vs baseline: 206.7675x; 1.1247x over previous
import functools

import jax
import jax.numpy as jnp
from jax import lax
from jax.experimental import pallas as pl
from jax.experimental.pallas import tpu as pltpu
from jax.experimental.pallas import tpu_sc as plsc

SHAPE = (8192, 2048)
TOTAL = SHAPE[0] * SHAPE[1]
L = 16
C = 16384
RPC = C // SHAPE[1]
W = C + 2 * L
NBLK = TOTAL // C
NWORK = 32
CPW = NBLK // NWORK
NBG = (CPW + 1 + L - 1) // L + 1


def _sc_scatter(params, tail_idx, tail_val, idx_p, val_p, npad):
    clamp = max(0, (npad - W)) & ~7
    steps = int(npad).bit_length() + 1

    mesh = plsc.VectorSubcoreMesh(core_axis_name="c", subcore_axis_name="s",
                                  num_cores=2, num_subcores=16)

    @functools.partial(
        pl.kernel,
        out_type=jax.ShapeDtypeStruct(SHAPE, jnp.float32),
        mesh=mesh,
        compiler_params=pltpu.CompilerParams(needs_layout_passes=False),
        scratch_types=[
            pltpu.VMEM((2 * L,), jnp.float32),
            pltpu.VMEM((NBG * L,), jnp.int32),
            pltpu.VMEM((NBG * L,), jnp.int32),
            pltpu.VMEM((NBG * L,), jnp.int32),
            pltpu.VMEM((L,), jnp.int32),
            pltpu.VMEM((L,), jnp.float32),
            pltpu.VMEM((W,), jnp.int32),
            pltpu.VMEM((W,), jnp.int32),
            pltpu.VMEM((W,), jnp.float32),
            pltpu.VMEM((W,), jnp.float32),
            pltpu.VMEM((RPC, SHAPE[1]), jnp.float32),
            pltpu.VMEM((RPC, SHAPE[1]), jnp.float32),
            pltpu.SemaphoreType.DMA,
            pltpu.SemaphoreType.DMA,
            pltpu.SemaphoreType.DMA,
            pltpu.SemaphoreType.DMA,
            pltpu.SemaphoreType.DMA,
        ],
    )
    def k(params_h, tidx_h, tval_h, idx_h, val_h, out_h,
          params_v, starts_v, probe_v, pdst_v, tidx_v, tval_v,
          idxw0, idxw1, valw0, valw1, buf0, buf1,
          psem, wsem0, wsem1, osem0, osem1):
        idxw = (idxw0, idxw1)
        valw = (valw0, valw1)
        buf = (buf0, buf1)
        wsem = (wsem0, wsem1)
        osem = (osem0, osem1)

        wid = lax.axis_index("s") * 2 + lax.axis_index("c")
        pltpu.sync_copy(params_h, params_v)
        pltpu.sync_copy(tidx_h, tidx_v)
        pltpu.sync_copy(tval_h, tval_v)
        scale_vec = jnp.exp(params_v[pl.ds(0, L)])
        zvec = params_v[pl.ds(L, L)]
        tiv = tidx_v[...]
        tvv = tval_v[...] * scale_vec
        cu = jnp.full((L,), C, jnp.uint32)

        lane = lax.iota(jnp.int32, L)
        base_b = wid * CPW
        bval = []
        for g in range(NBG):
            bval.append((base_b + g * L + lane) * C)
        los = [jnp.zeros((L,), jnp.int32)] * NBG
        his = [jnp.full((L,), npad, jnp.int32)] * NBG

        def search_body(_, carry):
            los, his = carry
            mids = []
            for g in range(NBG):
                mid = (los[g] + his[g]) >> 1
                mids.append(mid)
                probe_v[pl.ds(g * L, L)] = jnp.minimum(
                    mid, jnp.int32(npad - 1))
            cp = pltpu.async_copy(
                idx_h.at[probe_v], pdst_v, psem)
            cp.wait()
            nlos, nhis = [], []
            for g in range(NBG):
                gv = pdst_v[pl.ds(g * L, L)]
                active = los[g] < his[g]
                less = gv < bval[g]
                nlos.append(jnp.where(active & less, mids[g] + 1, los[g]))
                nhis.append(jnp.where(active & (~less), mids[g], his[g]))
            return nlos, nhis

        los, his = lax.fori_loop(0, steps, search_body, (los, his))
        for g in range(NBG):
            starts_v[pl.ds(g * L, L)] = his[g]

        def win_a8(k_):
            sv = starts_v[pl.ds(k_, L)]
            a8 = pl.multiple_of(jnp.minimum(sv[0] & ~7, clamp), 8)
            return sv, a8

        def start_win(k_, p):
            _, a8 = win_a8(k_)
            pltpu.async_copy(idx_h.at[pl.ds(a8, W)], idxw[p], wsem[p])
            pltpu.async_copy(val_h.at[pl.ds(a8, W)], valw[p], wsem[p])

        def out_slice(k_):
            r0 = pl.multiple_of((wid * CPW + k_) * RPC, RPC)
            return out_h.at[pl.ds(r0, RPC), :]

        start_win(0, 0)

        def body(kk2, carry):
            for p in (0, 1):
                k_ = 2 * kk2 + p
                sv, a8 = win_a8(k_)
                lo = (wid * CPW + k_) * C
                b = sv[1]
                start_win(k_ + 1, 1 - p)

                @pl.when(kk2 > 0)
                def _():
                    pltpu.make_async_copy(
                        buf[p], out_slice(0),
                        osem[p]).wait()

                for r in range(RPC):
                    rowv = jnp.full((L,), r, jnp.int32)

                    def zbody(j, colv, rowv=rowv):
                        plsc.store_scatter(buf[p], [rowv, colv], zvec)
                        return colv + jnp.int32(L)
                    lax.fori_loop(0, SHAPE[1] // L, zbody, lane, unroll=8)

                pltpu.make_async_copy(
                    idx_h.at[pl.ds(a8, W)], idxw[p], wsem[p]).wait()
                pltpu.make_async_copy(
                    val_h.at[pl.ds(a8, W)], valw[p], wsem[p]).wait()

                ng = (b - a8 + (L - 1)) // L
                lov = jnp.full((L,), lo, jnp.int32)

                def sbody(j, c):
                    iv = idxw[p][pl.ds(j * L, L)]
                    vv = valw[p][pl.ds(j * L, L)]
                    rel = iv - lov
                    m = plsc.bitcast(rel, jnp.uint32) < cu
                    row = lax.shift_right_logical(rel, 11)
                    col = rel & jnp.int32(SHAPE[1] - 1)
                    plsc.store_scatter(
                        buf[p], [row, col], vv * scale_vec, mask=m)
                    return c
                lax.fori_loop(0, ng, sbody, 0)

                trel = tiv - lov
                tm = plsc.bitcast(trel, jnp.uint32) < cu
                trow = lax.shift_right_logical(trel, 11)
                tcol = trel & jnp.int32(SHAPE[1] - 1)
                plsc.store_scatter(buf[p], [trow, tcol], tvv, mask=tm)

                pltpu.async_copy(
                    buf[p], out_slice(k_), osem[p])
            return carry

        lax.fori_loop(0, CPW // 2, body, 0)

        pltpu.make_async_copy(idx_h.at[pl.ds(0, W)], idxw[0], wsem[0]).wait()
        pltpu.make_async_copy(val_h.at[pl.ds(0, W)], valw[0], wsem[0]).wait()
        pltpu.make_async_copy(
            buf[0], out_slice(0), osem[0]).wait()
        pltpu.make_async_copy(
            buf[1], out_slice(0), osem[1]).wait()

    return k(params, tail_idx, tail_val, idx_p, val_p)


def kernel(log_scale, unscaled_x, idx, total_slots):
    n = idx.shape[0]
    zofs = (jnp.asarray(total_slots, jnp.float32) - jnp.float32(TOTAL))
    params = jnp.concatenate(
        [jnp.full((L,), jnp.asarray(log_scale, jnp.float32)),
         jnp.full((L,), zofs)])

    idx = idx.astype(jnp.int32)
    vals = unscaled_x.astype(jnp.float32)
    if n < W:
        pad = W - n
        idx_p = jnp.concatenate([idx, jnp.full((pad,), TOTAL, jnp.int32)])
        val_p = jnp.concatenate([vals, jnp.zeros((pad,), jnp.float32)])
        npad = W
    else:
        idx_p, val_p, npad = idx, vals, n

    t = max(0, n - L)
    tail_idx = lax.dynamic_slice_in_dim(idx_p, min(t, npad - L), L)
    tail_val = lax.dynamic_slice_in_dim(val_p, min(t, npad - L), L)

    return _sc_scatter(params, tail_idx, tail_val, idx_p, val_p, npad)


# speedup vs baseline: 224.2603x; 1.0846x over previous
import functools

import jax
import jax.numpy as jnp
from jax import lax
from jax.experimental import pallas as pl
from jax.experimental.pallas import tpu as pltpu
from jax.experimental.pallas import tpu_sc as plsc

SHAPE = (8192, 2048)
TOTAL = SHAPE[0] * SHAPE[1]
L = 16
C = 16384
RPC = C // SHAPE[1]
W = C + 2 * L
NBLK = TOTAL // C
NWORK = 32
CPW = NBLK // NWORK
NBG = (CPW + 1 + L - 1) // L + 1


def _sc_scatter(params, tail_idx, tail_val, idx_p, val_p, npad):
    clamp = max(0, (npad - W)) & ~7
    steps = int(npad).bit_length() + 1

    mesh = plsc.VectorSubcoreMesh(core_axis_name="c", subcore_axis_name="s",
                                  num_cores=2, num_subcores=16)

    @functools.partial(
        pl.kernel,
        out_type=jax.ShapeDtypeStruct(SHAPE, jnp.float32),
        mesh=mesh,
        compiler_params=pltpu.CompilerParams(needs_layout_passes=False),
        scratch_types=[
            pltpu.VMEM((2 * L,), jnp.float32),
            pltpu.VMEM((NBG * L,), jnp.int32),
            pltpu.VMEM((NBG * L,), jnp.int32),
            pltpu.VMEM((NBG * L,), jnp.int32),
            pltpu.VMEM((L,), jnp.int32),
            pltpu.VMEM((L,), jnp.float32),
            pltpu.VMEM((W,), jnp.int32),
            pltpu.VMEM((W,), jnp.int32),
            pltpu.VMEM((W,), jnp.float32),
            pltpu.VMEM((W,), jnp.float32),
            pltpu.VMEM((RPC, SHAPE[1]), jnp.float32),
            pltpu.VMEM((RPC, SHAPE[1]), jnp.float32),
            pltpu.SemaphoreType.DMA,
            pltpu.SemaphoreType.DMA,
            pltpu.SemaphoreType.DMA,
            pltpu.SemaphoreType.DMA,
            pltpu.SemaphoreType.DMA,
        ],
    )
    def k(params_h, tidx_h, tval_h, idx_h, val_h, out_h,
          params_v, starts_v, probe_v, pdst_v, tidx_v, tval_v,
          idxw0, idxw1, valw0, valw1, buf0, buf1,
          psem, wsem0, wsem1, osem0, osem1):
        idxw = (idxw0, idxw1)
        valw = (valw0, valw1)
        buf = (buf0, buf1)
        wsem = (wsem0, wsem1)
        osem = (osem0, osem1)

        wid = lax.axis_index("s") * 2 + lax.axis_index("c")
        pltpu.sync_copy(params_h, params_v)
        pltpu.sync_copy(tidx_h, tidx_v)
        pltpu.sync_copy(tval_h, tval_v)
        scale_vec = jnp.exp(params_v[pl.ds(0, L)])
        zvec = params_v[pl.ds(L, L)]
        tiv = tidx_v[...]
        tvv = tval_v[...] * scale_vec
        cu = jnp.full((L,), C, jnp.uint32)

        lane = lax.iota(jnp.int32, L)
        base_b = wid * CPW
        bval = []
        for g in range(NBG):
            bval.append((base_b + g * L + lane) * C)
        los = [jnp.zeros((L,), jnp.int32)] * NBG
        his = [jnp.full((L,), npad, jnp.int32)] * NBG

        def search_body(_, carry):
            los, his = carry
            mids = []
            for g in range(NBG):
                mid = (los[g] + his[g]) >> 1
                mids.append(mid)
                probe_v[pl.ds(g * L, L)] = jnp.minimum(
                    mid, jnp.int32(npad - 1))
            cp = pltpu.async_copy(
                idx_h.at[probe_v], pdst_v, psem)
            cp.wait()
            nlos, nhis = [], []
            for g in range(NBG):
                gv = pdst_v[pl.ds(g * L, L)]
                active = los[g] < his[g]
                less = gv < bval[g]
                nlos.append(jnp.where(active & less, mids[g] + 1, los[g]))
                nhis.append(jnp.where(active & (~less), mids[g], his[g]))
            return nlos, nhis

        los, his = lax.fori_loop(0, steps, search_body, (los, his))
        for g in range(NBG):
            starts_v[pl.ds(g * L, L)] = his[g]

        def win_a8(k_):
            sv = starts_v[pl.ds(k_, L)]
            a8 = pl.multiple_of(jnp.minimum(sv[0] & ~7, clamp), 8)
            return sv, a8

        def start_win(k_, p):
            _, a8 = win_a8(k_)
            pltpu.async_copy(idx_h.at[pl.ds(a8, W)], idxw[p], wsem[p])
            pltpu.async_copy(val_h.at[pl.ds(a8, W)], valw[p], wsem[p])

        def out_slice(k_):
            r0 = pl.multiple_of((wid * CPW + k_) * RPC, RPC)
            return out_h.at[pl.ds(r0, RPC), :]

        start_win(0, 0)

        def body(kk2, carry):
            for p in (0, 1):
                k_ = 2 * kk2 + p
                sv, a8 = win_a8(k_)
                lo = (wid * CPW + k_) * C
                b = sv[1]
                start_win(k_ + 1, 1 - p)

                @pl.when(kk2 > 0)
                def _():
                    pltpu.make_async_copy(
                        buf[p], out_slice(0),
                        osem[p]).wait()

                for r in range(RPC):
                    def zbody(j, c, r=r):
                        buf[p][r, pl.ds(j * L, L)] = zvec
                        return c
                    lax.fori_loop(0, SHAPE[1] // L, zbody, 0, unroll=8)

                pltpu.make_async_copy(
                    idx_h.at[pl.ds(a8, W)], idxw[p], wsem[p]).wait()
                pltpu.make_async_copy(
                    val_h.at[pl.ds(a8, W)], valw[p], wsem[p]).wait()

                ng = (b - a8 + (L - 1)) // L
                lov = jnp.full((L,), lo, jnp.int32)

                def sbody(j, c):
                    iv = idxw[p][pl.ds(j * L, L)]
                    vv = valw[p][pl.ds(j * L, L)]
                    rel = iv - lov
                    m = plsc.bitcast(rel, jnp.uint32) < cu
                    row = lax.shift_right_logical(rel, 11)
                    col = rel & jnp.int32(SHAPE[1] - 1)
                    plsc.store_scatter(
                        buf[p], [row, col], vv * scale_vec, mask=m)
                    return c
                lax.fori_loop(0, ng, sbody, 0)

                trel = tiv - lov
                tm = plsc.bitcast(trel, jnp.uint32) < cu
                trow = lax.shift_right_logical(trel, 11)
                tcol = trel & jnp.int32(SHAPE[1] - 1)
                plsc.store_scatter(buf[p], [trow, tcol], tvv, mask=tm)

                pltpu.async_copy(
                    buf[p], out_slice(k_), osem[p])
            return carry

        lax.fori_loop(0, CPW // 2, body, 0)

        pltpu.make_async_copy(idx_h.at[pl.ds(0, W)], idxw[0], wsem[0]).wait()
        pltpu.make_async_copy(val_h.at[pl.ds(0, W)], valw[0], wsem[0]).wait()
        pltpu.make_async_copy(
            buf[0], out_slice(0), osem[0]).wait()
        pltpu.make_async_copy(
            buf[1], out_slice(0), osem[1]).wait()

    return k(params, tail_idx, tail_val, idx_p, val_p)


def kernel(log_scale, unscaled_x, idx, total_slots):
    n = idx.shape[0]
    zofs = (jnp.asarray(total_slots, jnp.float32) - jnp.float32(TOTAL))
    params = jnp.concatenate(
        [jnp.full((L,), jnp.asarray(log_scale, jnp.float32)),
         jnp.full((L,), zofs)])

    idx = idx.astype(jnp.int32)
    vals = unscaled_x.astype(jnp.float32)
    if n < W:
        pad = W - n
        idx_p = jnp.concatenate([idx, jnp.full((pad,), TOTAL, jnp.int32)])
        val_p = jnp.concatenate([vals, jnp.zeros((pad,), jnp.float32)])
        npad = W
    else:
        idx_p, val_p, npad = idx, vals, n

    t = max(0, n - L)
    tail_idx = lax.dynamic_slice_in_dim(idx_p, min(t, npad - L), L)
    tail_val = lax.dynamic_slice_in_dim(val_p, min(t, npad - L), L)

    return _sc_scatter(params, tail_idx, tail_val, idx_p, val_p, npad)


# speedup vs baseline: 225.3104x; 1.0047x over previous
import functools

import jax
import jax.numpy as jnp
from jax import lax
from jax.experimental import pallas as pl
from jax.experimental.pallas import tpu as pltpu
from jax.experimental.pallas import tpu_sc as plsc

SHAPE = (8192, 2048)
TOTAL = SHAPE[0] * SHAPE[1]
L = 16
C = 16384
RPC = C // SHAPE[1]
W = C + 2 * L
WS = 5 * C // 8 + 2 * L
NBLK = TOTAL // C
NWORK = 32
CPW = NBLK // NWORK
NBG = (CPW + 1 + L - 1) // L + 1


def _sc_scatter(params, tail_idx, tail_val, idx_p, val_p, npad):
    clamp = max(0, (npad - W)) & ~7
    steps = int(npad).bit_length() + 1

    mesh = plsc.VectorSubcoreMesh(core_axis_name="c", subcore_axis_name="s",
                                  num_cores=2, num_subcores=16)

    @functools.partial(
        pl.kernel,
        out_type=jax.ShapeDtypeStruct(SHAPE, jnp.float32),
        mesh=mesh,
        compiler_params=pltpu.CompilerParams(needs_layout_passes=False),
        scratch_types=[
            pltpu.VMEM((2 * L,), jnp.float32),
            pltpu.VMEM((NBG * L,), jnp.int32),
            pltpu.VMEM((NBG * L,), jnp.int32),
            pltpu.VMEM((NBG * L,), jnp.int32),
            pltpu.VMEM((L,), jnp.int32),
            pltpu.VMEM((L,), jnp.float32),
            pltpu.VMEM((W,), jnp.int32),
            pltpu.VMEM((W,), jnp.int32),
            pltpu.VMEM((W,), jnp.float32),
            pltpu.VMEM((W,), jnp.float32),
            pltpu.VMEM((RPC, SHAPE[1]), jnp.float32),
            pltpu.VMEM((RPC, SHAPE[1]), jnp.float32),
            pltpu.SemaphoreType.DMA,
            pltpu.SemaphoreType.DMA,
            pltpu.SemaphoreType.DMA,
            pltpu.SemaphoreType.DMA,
            pltpu.SemaphoreType.DMA,
        ],
    )
    def k(params_h, tidx_h, tval_h, idx_h, val_h, out_h,
          params_v, starts_v, probe_v, pdst_v, tidx_v, tval_v,
          idxw0, idxw1, valw0, valw1, buf0, buf1,
          psem, wsem0, wsem1, osem0, osem1):
        idxw = (idxw0, idxw1)
        valw = (valw0, valw1)
        buf = (buf0, buf1)
        wsem = (wsem0, wsem1)
        osem = (osem0, osem1)

        wid = lax.axis_index("s") * 2 + lax.axis_index("c")
        pltpu.sync_copy(params_h, params_v)
        pltpu.sync_copy(tidx_h, tidx_v)
        pltpu.sync_copy(tval_h, tval_v)
        scale_vec = jnp.exp(params_v[pl.ds(0, L)])
        zvec = params_v[pl.ds(L, L)]
        tiv = tidx_v[...]
        tvv = tval_v[...] * scale_vec
        cu = jnp.full((L,), C, jnp.uint32)

        lane = lax.iota(jnp.int32, L)
        base_b = wid * CPW
        bval = []
        for g in range(NBG):
            bval.append((base_b + g * L + lane) * C)
        los = [jnp.zeros((L,), jnp.int32)] * NBG
        his = [jnp.full((L,), npad, jnp.int32)] * NBG

        def search_body(_, carry):
            los, his = carry
            mids = []
            for g in range(NBG):
                mid = (los[g] + his[g]) >> 1
                mids.append(mid)
                probe_v[pl.ds(g * L, L)] = jnp.minimum(
                    mid, jnp.int32(npad - 1))
            cp = pltpu.async_copy(
                idx_h.at[probe_v], pdst_v, psem)
            cp.wait()
            nlos, nhis = [], []
            for g in range(NBG):
                gv = pdst_v[pl.ds(g * L, L)]
                active = los[g] < his[g]
                less = gv < bval[g]
                nlos.append(jnp.where(active & less, mids[g] + 1, los[g]))
                nhis.append(jnp.where(active & (~less), mids[g], his[g]))
            return nlos, nhis

        los, his = lax.fori_loop(0, steps, search_body, (los, his))
        for g in range(NBG):
            starts_v[pl.ds(g * L, L)] = his[g]

        def win_a8(k_):
            sv = starts_v[pl.ds(k_, L)]
            a8 = pl.multiple_of(jnp.minimum(sv[0] & ~7, clamp), 8)
            return sv, a8

        def win_small(sv, a8):
            return (sv[1] - a8) <= (WS - L)

        def start_win(k_, p):
            sv, a8 = win_a8(k_)
            small = win_small(sv, a8)

            @pl.when(small)
            def _():
                pltpu.async_copy(idx_h.at[pl.ds(a8, WS)],
                                 idxw[p].at[pl.ds(0, WS)], wsem[p])
                pltpu.async_copy(val_h.at[pl.ds(a8, WS)],
                                 valw[p].at[pl.ds(0, WS)], wsem[p])

            @pl.when(jnp.logical_not(small))
            def _():
                pltpu.async_copy(idx_h.at[pl.ds(a8, W)], idxw[p], wsem[p])
                pltpu.async_copy(val_h.at[pl.ds(a8, W)], valw[p], wsem[p])

        def out_slice(k_):
            r0 = pl.multiple_of((wid * CPW + k_) * RPC, RPC)
            return out_h.at[pl.ds(r0, RPC), :]

        start_win(0, 0)

        def body(kk2, carry):
            for p in (0, 1):
                k_ = 2 * kk2 + p
                sv, a8 = win_a8(k_)
                lo = (wid * CPW + k_) * C
                b = sv[1]
                small = win_small(sv, a8)
                start_win(k_ + 1, 1 - p)

                @pl.when(kk2 > 0)
                def _():
                    pltpu.make_async_copy(
                        buf[p], out_slice(0),
                        osem[p]).wait()

                for r in range(RPC):
                    def zbody(j, c, r=r):
                        buf[p][r, pl.ds(j * L, L)] = zvec
                        return c
                    lax.fori_loop(0, SHAPE[1] // L, zbody, 0, unroll=8)

                @pl.when(small)
                def _():
                    pltpu.make_async_copy(
                        idx_h.at[pl.ds(a8, WS)],
                        idxw[p].at[pl.ds(0, WS)], wsem[p]).wait()
                    pltpu.make_async_copy(
                        val_h.at[pl.ds(a8, WS)],
                        valw[p].at[pl.ds(0, WS)], wsem[p]).wait()

                @pl.when(jnp.logical_not(small))
                def _():
                    pltpu.make_async_copy(
                        idx_h.at[pl.ds(a8, W)], idxw[p], wsem[p]).wait()
                    pltpu.make_async_copy(
                        val_h.at[pl.ds(a8, W)], valw[p], wsem[p]).wait()

                ng = (b - a8 + (L - 1)) // L
                lov = jnp.full((L,), lo, jnp.int32)

                def sbody(j, c):
                    iv = idxw[p][pl.ds(j * L, L)]
                    vv = valw[p][pl.ds(j * L, L)]
                    rel = iv - lov
                    m = plsc.bitcast(rel, jnp.uint32) < cu
                    row = lax.shift_right_logical(rel, 11)
                    col = rel & jnp.int32(SHAPE[1] - 1)
                    plsc.store_scatter(
                        buf[p], [row, col], vv * scale_vec, mask=m)
                    return c
                lax.fori_loop(0, ng, sbody, 0)

                trel = tiv - lov
                tm = plsc.bitcast(trel, jnp.uint32) < cu
                trow = lax.shift_right_logical(trel, 11)
                tcol = trel & jnp.int32(SHAPE[1] - 1)
                plsc.store_scatter(buf[p], [trow, tcol], tvv, mask=tm)

                pltpu.async_copy(
                    buf[p], out_slice(k_), osem[p])
            return carry

        lax.fori_loop(0, CPW // 2, body, 0)

        svd, a8d = win_a8(CPW)
        smalld = win_small(svd, a8d)

        @pl.when(smalld)
        def _():
            pltpu.make_async_copy(
                idx_h.at[pl.ds(a8d, WS)], idxw[0].at[pl.ds(0, WS)],
                wsem[0]).wait()
            pltpu.make_async_copy(
                val_h.at[pl.ds(a8d, WS)], valw[0].at[pl.ds(0, WS)],
                wsem[0]).wait()

        @pl.when(jnp.logical_not(smalld))
        def _():
            pltpu.make_async_copy(
                idx_h.at[pl.ds(0, W)], idxw[0], wsem[0]).wait()
            pltpu.make_async_copy(
                val_h.at[pl.ds(0, W)], valw[0], wsem[0]).wait()
        pltpu.make_async_copy(
            buf[0], out_slice(0), osem[0]).wait()
        pltpu.make_async_copy(
            buf[1], out_slice(0), osem[1]).wait()

    return k(params, tail_idx, tail_val, idx_p, val_p)


def kernel(log_scale, unscaled_x, idx, total_slots):
    n = idx.shape[0]
    zofs = (jnp.asarray(total_slots, jnp.float32) - jnp.float32(TOTAL))
    params = jnp.concatenate(
        [jnp.full((L,), jnp.asarray(log_scale, jnp.float32)),
         jnp.full((L,), zofs)])

    idx = idx.astype(jnp.int32)
    vals = unscaled_x.astype(jnp.float32)
    if n < W:
        pad = W - n
        idx_p = jnp.concatenate([idx, jnp.full((pad,), TOTAL, jnp.int32)])
        val_p = jnp.concatenate([vals, jnp.zeros((pad,), jnp.float32)])
        npad = W
    else:
        idx_p, val_p, npad = idx, vals, n

    t = max(0, n - L)
    tail_idx = lax.dynamic_slice_in_dim(idx_p, min(t, npad - L), L)
    tail_val = lax.dynamic_slice_in_dim(val_p, min(t, npad - L), L)

    return _sc_scatter(params, tail_idx, tail_val, idx_p, val_p, npad)


# speedup vs baseline: 229.9714x; 1.0207x over previous
import functools

import jax
import jax.numpy as jnp
from jax import lax
from jax.experimental import pallas as pl
from jax.experimental.pallas import tpu as pltpu
from jax.experimental.pallas import tpu_sc as plsc

SHAPE = (8192, 2048)
TOTAL = SHAPE[0] * SHAPE[1]
L = 16
C = 16384
RPC = C // SHAPE[1]
W = C + 3 * L
WS = 5 * C // 8 + 2 * L
NBLK = TOTAL // C
NWORK = 32
CPW = NBLK // NWORK
NBG = (CPW + 1 + L - 1) // L + 1


def _sc_scatter(params, tail_idx, tail_val, idx_p, val_p, npad):
    clamp = max(0, (npad - W)) & ~7
    steps = int(npad).bit_length() + 1

    mesh = plsc.VectorSubcoreMesh(core_axis_name="c", subcore_axis_name="s",
                                  num_cores=2, num_subcores=16)

    @functools.partial(
        pl.kernel,
        out_type=jax.ShapeDtypeStruct(SHAPE, jnp.float32),
        mesh=mesh,
        compiler_params=pltpu.CompilerParams(needs_layout_passes=False),
        scratch_types=[
            pltpu.VMEM((2 * L,), jnp.float32),
            pltpu.VMEM((NBG * L,), jnp.int32),
            pltpu.VMEM((NBG * L,), jnp.int32),
            pltpu.VMEM((NBG * L,), jnp.int32),
            pltpu.VMEM((L,), jnp.int32),
            pltpu.VMEM((L,), jnp.float32),
            pltpu.VMEM((W,), jnp.int32),
            pltpu.VMEM((W,), jnp.int32),
            pltpu.VMEM((W,), jnp.float32),
            pltpu.VMEM((W,), jnp.float32),
            pltpu.VMEM((RPC, SHAPE[1]), jnp.float32),
            pltpu.VMEM((RPC, SHAPE[1]), jnp.float32),
            pltpu.SemaphoreType.DMA,
            pltpu.SemaphoreType.DMA,
            pltpu.SemaphoreType.DMA,
            pltpu.SemaphoreType.DMA,
            pltpu.SemaphoreType.DMA,
        ],
    )
    def k(params_h, tidx_h, tval_h, idx_h, val_h, out_h,
          params_v, starts_v, probe_v, pdst_v, tidx_v, tval_v,
          idxw0, idxw1, valw0, valw1, buf0, buf1,
          psem, wsem0, wsem1, osem0, osem1):
        idxw = (idxw0, idxw1)
        valw = (valw0, valw1)
        buf = (buf0, buf1)
        wsem = (wsem0, wsem1)
        osem = (osem0, osem1)

        wid = lax.axis_index("s") * 2 + lax.axis_index("c")
        pltpu.sync_copy(params_h, params_v)
        pltpu.sync_copy(tidx_h, tidx_v)
        pltpu.sync_copy(tval_h, tval_v)
        scale_vec = jnp.exp(params_v[pl.ds(0, L)])
        zvec = params_v[pl.ds(L, L)]
        tiv = tidx_v[...]
        tvv = tval_v[...] * scale_vec
        cu = jnp.full((L,), C, jnp.uint32)

        lane = lax.iota(jnp.int32, L)
        base_b = wid * CPW
        bval = []
        for g in range(NBG):
            bval.append((base_b + g * L + lane) * C)
        los = [jnp.zeros((L,), jnp.int32)] * NBG
        his = [jnp.full((L,), npad, jnp.int32)] * NBG

        def search_body(_, carry):
            los, his = carry
            mids = []
            for g in range(NBG):
                mid = (los[g] + his[g]) >> 1
                mids.append(mid)
                probe_v[pl.ds(g * L, L)] = jnp.minimum(
                    mid, jnp.int32(npad - 1))
            cp = pltpu.async_copy(
                idx_h.at[probe_v], pdst_v, psem)
            cp.wait()
            nlos, nhis = [], []
            for g in range(NBG):
                gv = pdst_v[pl.ds(g * L, L)]
                active = los[g] < his[g]
                less = gv < bval[g]
                nlos.append(jnp.where(active & less, mids[g] + 1, los[g]))
                nhis.append(jnp.where(active & (~less), mids[g], his[g]))
            return nlos, nhis

        los, his = lax.fori_loop(0, steps, search_body, (los, his))
        for g in range(NBG):
            starts_v[pl.ds(g * L, L)] = his[g]

        def win_a8(k_):
            sv = starts_v[pl.ds(k_, L)]
            a8 = pl.multiple_of(jnp.minimum(sv[0] & ~7, clamp), 8)
            return sv, a8

        def win_small(sv, a8):
            return (sv[1] - a8) <= (WS - 2 * L)

        def start_win(k_, p):
            sv, a8 = win_a8(k_)
            small = win_small(sv, a8)

            @pl.when(small)
            def _():
                pltpu.async_copy(idx_h.at[pl.ds(a8, WS)],
                                 idxw[p].at[pl.ds(0, WS)], wsem[p])
                pltpu.async_copy(val_h.at[pl.ds(a8, WS)],
                                 valw[p].at[pl.ds(0, WS)], wsem[p])

            @pl.when(jnp.logical_not(small))
            def _():
                pltpu.async_copy(idx_h.at[pl.ds(a8, W)], idxw[p], wsem[p])
                pltpu.async_copy(val_h.at[pl.ds(a8, W)], valw[p], wsem[p])

        def out_slice(k_):
            r0 = pl.multiple_of((wid * CPW + k_) * RPC, RPC)
            return out_h.at[pl.ds(r0, RPC), :]

        start_win(0, 0)

        def body(kk2, carry):
            for p in (0, 1):
                k_ = 2 * kk2 + p
                sv, a8 = win_a8(k_)
                lo = (wid * CPW + k_) * C
                b = sv[1]
                small = win_small(sv, a8)
                start_win(k_ + 1, 1 - p)

                @pl.when(kk2 > 0)
                def _():
                    pltpu.make_async_copy(
                        buf[p], out_slice(0),
                        osem[p]).wait()

                for r in range(RPC):
                    def zbody(j, c, r=r):
                        buf[p][r, pl.ds(j * L, L)] = zvec
                        return c
                    lax.fori_loop(0, SHAPE[1] // L, zbody, 0, unroll=8)

                @pl.when(small)
                def _():
                    pltpu.make_async_copy(
                        idx_h.at[pl.ds(a8, WS)],
                        idxw[p].at[pl.ds(0, WS)], wsem[p]).wait()
                    pltpu.make_async_copy(
                        val_h.at[pl.ds(a8, WS)],
                        valw[p].at[pl.ds(0, WS)], wsem[p]).wait()

                @pl.when(jnp.logical_not(small))
                def _():
                    pltpu.make_async_copy(
                        idx_h.at[pl.ds(a8, W)], idxw[p], wsem[p]).wait()
                    pltpu.make_async_copy(
                        val_h.at[pl.ds(a8, W)], valw[p], wsem[p]).wait()

                ng2 = (b - a8 + (2 * L - 1)) // (2 * L)
                lov = jnp.full((L,), lo, jnp.int32)

                def sbody(j, c):
                    for u in range(2):
                        off = j * 2 * L + u * L
                        iv = idxw[p][pl.ds(off, L)]
                        vv = valw[p][pl.ds(off, L)]
                        rel = iv - lov
                        m = plsc.bitcast(rel, jnp.uint32) < cu
                        row = lax.shift_right_logical(rel, 11)
                        col = rel & jnp.int32(SHAPE[1] - 1)
                        plsc.store_scatter(
                            buf[p], [row, col], vv * scale_vec, mask=m)
                    return c
                lax.fori_loop(0, ng2, sbody, 0)

                trel = tiv - lov
                tm = plsc.bitcast(trel, jnp.uint32) < cu
                trow = lax.shift_right_logical(trel, 11)
                tcol = trel & jnp.int32(SHAPE[1] - 1)
                plsc.store_scatter(buf[p], [trow, tcol], tvv, mask=tm)

                pltpu.async_copy(
                    buf[p], out_slice(k_), osem[p])
            return carry

        lax.fori_loop(0, CPW // 2, body, 0)

        svd, a8d = win_a8(CPW)
        smalld = win_small(svd, a8d)

        @pl.when(smalld)
        def _():
            pltpu.make_async_copy(
                idx_h.at[pl.ds(a8d, WS)], idxw[0].at[pl.ds(0, WS)],
                wsem[0]).wait()
            pltpu.make_async_copy(
                val_h.at[pl.ds(a8d, WS)], valw[0].at[pl.ds(0, WS)],
                wsem[0]).wait()

        @pl.when(jnp.logical_not(smalld))
        def _():
            pltpu.make_async_copy(
                idx_h.at[pl.ds(0, W)], idxw[0], wsem[0]).wait()
            pltpu.make_async_copy(
                val_h.at[pl.ds(0, W)], valw[0], wsem[0]).wait()
        pltpu.make_async_copy(
            buf[0], out_slice(0), osem[0]).wait()
        pltpu.make_async_copy(
            buf[1], out_slice(0), osem[1]).wait()

    return k(params, tail_idx, tail_val, idx_p, val_p)


def kernel(log_scale, unscaled_x, idx, total_slots):
    n = idx.shape[0]
    zofs = (jnp.asarray(total_slots, jnp.float32) - jnp.float32(TOTAL))
    params = jnp.concatenate(
        [jnp.full((L,), jnp.asarray(log_scale, jnp.float32)),
         jnp.full((L,), zofs)])

    idx = idx.astype(jnp.int32)
    vals = unscaled_x.astype(jnp.float32)
    if n < W:
        pad = W - n
        idx_p = jnp.concatenate([idx, jnp.full((pad,), TOTAL, jnp.int32)])
        val_p = jnp.concatenate([vals, jnp.zeros((pad,), jnp.float32)])
        npad = W
    else:
        idx_p, val_p, npad = idx, vals, n

    t = max(0, n - L)
    tail_idx = lax.dynamic_slice_in_dim(idx_p, min(t, npad - L), L)
    tail_val = lax.dynamic_slice_in_dim(val_p, min(t, npad - L), L)

    return _sc_scatter(params, tail_idx, tail_val, idx_p, val_p, npad)


# speedup vs baseline: 495.6679x; 2.1553x over previous
import functools

import jax
import jax.numpy as jnp
from jax import lax
from jax.experimental import pallas as pl
from jax.experimental.pallas import tpu as pltpu
from jax.experimental.pallas import tpu_sc as plsc

SHAPE = (8192, 2048)
TOTAL = SHAPE[0] * SHAPE[1]
L = 16
C = 16384
RPC = C // SHAPE[1]
W = C + 3 * L
WS = 5 * C // 8 + 2 * L
NBLK = TOTAL // C
NWORK = 32
CPW = NBLK // NWORK
NBG = (CPW + 1 + L - 1) // L + 1


def _sc_scatter(params, tail_idx, tail_val, idx_p, val_p, npad):
    clamp = max(0, (npad - W)) & ~7
    steps = int(npad).bit_length() + 1

    mesh = plsc.VectorSubcoreMesh(core_axis_name="c", subcore_axis_name="s",
                                  num_cores=2, num_subcores=16)

    @functools.partial(
        pl.kernel,
        out_type=jax.ShapeDtypeStruct(SHAPE, jnp.float32),
        mesh=mesh,
        compiler_params=pltpu.CompilerParams(needs_layout_passes=False),
        scratch_types=[
            pltpu.VMEM((2 * L,), jnp.float32),
            pltpu.VMEM((NBG * L,), jnp.int32),
            pltpu.VMEM((NBG * L,), jnp.int32),
            pltpu.VMEM((NBG * L,), jnp.int32),
            pltpu.VMEM((L,), jnp.int32),
            pltpu.VMEM((L,), jnp.float32),
            pltpu.VMEM((W,), jnp.int32),
            pltpu.VMEM((W,), jnp.int32),
            pltpu.VMEM((W,), jnp.float32),
            pltpu.VMEM((W,), jnp.float32),
            pltpu.VMEM((RPC, SHAPE[1]), jnp.float32),
            pltpu.VMEM((RPC, SHAPE[1]), jnp.float32),
            pltpu.SemaphoreType.DMA,
            pltpu.SemaphoreType.DMA,
            pltpu.SemaphoreType.DMA,
            pltpu.SemaphoreType.DMA,
            pltpu.SemaphoreType.DMA,
        ],
    )
    def k(params_h, tidx_h, tval_h, idx_h, val_h, out_h,
          params_v, starts_v, probe_v, pdst_v, tidx_v, tval_v,
          idxw0, idxw1, valw0, valw1, buf0, buf1,
          psem, wsem0, wsem1, osem0, osem1):
        idxw = (idxw0, idxw1)
        valw = (valw0, valw1)
        buf = (buf0, buf1)
        wsem = (wsem0, wsem1)
        osem = (osem0, osem1)

        wid = lax.axis_index("s") * 2 + lax.axis_index("c")
        pltpu.sync_copy(params_h, params_v)
        pltpu.sync_copy(tidx_h, tidx_v)
        pltpu.sync_copy(tval_h, tval_v)
        scale_vec = jnp.exp(params_v[pl.ds(0, L)])
        zvec = params_v[pl.ds(L, L)]
        tiv = tidx_v[...]
        tvv = tval_v[...] * scale_vec
        cu = jnp.full((L,), C, jnp.uint32)

        lane = lax.iota(jnp.int32, L)
        base_b = wid * CPW
        bval = []
        for g in range(NBG):
            bval.append((base_b + g * L + lane) * C)
        los = [jnp.zeros((L,), jnp.int32)] * NBG
        his = [jnp.full((L,), npad, jnp.int32)] * NBG

        def search_body(_, carry):
            los, his = carry
            mids = []
            for g in range(NBG):
                mid = (los[g] + his[g]) >> 1
                mids.append(mid)
                probe_v[pl.ds(g * L, L)] = jnp.minimum(
                    mid, jnp.int32(npad - 1))
            cp = pltpu.async_copy(
                idx_h.at[probe_v], pdst_v, psem)
            cp.wait()
            nlos, nhis = [], []
            for g in range(NBG):
                gv = pdst_v[pl.ds(g * L, L)]
                active = los[g] < his[g]
                less = gv < bval[g]
                nlos.append(jnp.where(active & less, mids[g] + 1, los[g]))
                nhis.append(jnp.where(active & (~less), mids[g], his[g]))
            return nlos, nhis

        los, his = lax.fori_loop(0, steps, search_body, (los, his))
        for g in range(NBG):
            starts_v[pl.ds(g * L, L)] = his[g]

        def win_a8(k_):
            sv = starts_v[pl.ds(k_, L)]
            a8 = pl.multiple_of(jnp.minimum(sv[0] & ~7, clamp), 8)
            return sv, a8

        def win_small(sv, a8):
            return (sv[1] - a8) <= (WS - 2 * L)

        def start_win(k_, p):
            sv, a8 = win_a8(k_)
            small = win_small(sv, a8)

            @pl.when(small)
            def _():
                pltpu.async_copy(idx_h.at[pl.ds(a8, WS)],
                                 idxw[p].at[pl.ds(0, WS)], wsem[p])
                pltpu.async_copy(val_h.at[pl.ds(a8, WS)],
                                 valw[p].at[pl.ds(0, WS)], wsem[p])

            @pl.when(jnp.logical_not(small))
            def _():
                pltpu.async_copy(idx_h.at[pl.ds(a8, W)], idxw[p], wsem[p])
                pltpu.async_copy(val_h.at[pl.ds(a8, W)], valw[p], wsem[p])

        def out_slice(k_):
            r0 = pl.multiple_of((wid * CPW + k_) * RPC, RPC)
            return out_h.at[pl.ds(r0, RPC), :]

        start_win(0, 0)

        def body(kk2, carry):
            for p in (0, 1):
                k_ = 2 * kk2 + p
                sv, a8 = win_a8(k_)
                lo = (wid * CPW + k_) * C
                b = sv[1]
                small = win_small(sv, a8)
                start_win(k_ + 1, 1 - p)

                @pl.when(kk2 > 0)
                def _():
                    pltpu.make_async_copy(
                        buf[p], out_slice(0),
                        osem[p]).wait()

                for r in range(RPC):
                    @functools.partial(
                        plsc.parallel_loop, 0, SHAPE[1] // L, unroll=8)
                    def _(j, r=r):
                        buf[p][r, pl.ds(j * L, L)] = zvec

                @pl.when(small)
                def _():
                    pltpu.make_async_copy(
                        idx_h.at[pl.ds(a8, WS)],
                        idxw[p].at[pl.ds(0, WS)], wsem[p]).wait()
                    pltpu.make_async_copy(
                        val_h.at[pl.ds(a8, WS)],
                        valw[p].at[pl.ds(0, WS)], wsem[p]).wait()

                @pl.when(jnp.logical_not(small))
                def _():
                    pltpu.make_async_copy(
                        idx_h.at[pl.ds(a8, W)], idxw[p], wsem[p]).wait()
                    pltpu.make_async_copy(
                        val_h.at[pl.ds(a8, W)], valw[p], wsem[p]).wait()

                ng2 = (b - a8 + (2 * L - 1)) // (2 * L)
                lov = jnp.full((L,), lo, jnp.int32)

                @functools.partial(plsc.parallel_loop, 0, ng2, unroll=2)
                def _(j):
                    for u in range(2):
                        off = j * 2 * L + u * L
                        iv = idxw[p][pl.ds(off, L)]
                        vv = valw[p][pl.ds(off, L)]
                        rel = iv - lov
                        m = plsc.bitcast(rel, jnp.uint32) < cu
                        row = lax.shift_right_logical(rel, 11)
                        col = rel & jnp.int32(SHAPE[1] - 1)
                        plsc.store_scatter(
                            buf[p], [row, col], vv * scale_vec, mask=m)

                trel = tiv - lov
                tm = plsc.bitcast(trel, jnp.uint32) < cu
                trow = lax.shift_right_logical(trel, 11)
                tcol = trel & jnp.int32(SHAPE[1] - 1)
                plsc.store_scatter(buf[p], [trow, tcol], tvv, mask=tm)

                pltpu.async_copy(
                    buf[p], out_slice(k_), osem[p])
            return carry

        lax.fori_loop(0, CPW // 2, body, 0)

        svd, a8d = win_a8(CPW)
        smalld = win_small(svd, a8d)

        @pl.when(smalld)
        def _():
            pltpu.make_async_copy(
                idx_h.at[pl.ds(a8d, WS)], idxw[0].at[pl.ds(0, WS)],
                wsem[0]).wait()
            pltpu.make_async_copy(
                val_h.at[pl.ds(a8d, WS)], valw[0].at[pl.ds(0, WS)],
                wsem[0]).wait()

        @pl.when(jnp.logical_not(smalld))
        def _():
            pltpu.make_async_copy(
                idx_h.at[pl.ds(0, W)], idxw[0], wsem[0]).wait()
            pltpu.make_async_copy(
                val_h.at[pl.ds(0, W)], valw[0], wsem[0]).wait()
        pltpu.make_async_copy(
            buf[0], out_slice(0), osem[0]).wait()
        pltpu.make_async_copy(
            buf[1], out_slice(0), osem[1]).wait()

    return k(params, tail_idx, tail_val, idx_p, val_p)


def kernel(log_scale, unscaled_x, idx, total_slots):
    n = idx.shape[0]
    zofs = (jnp.asarray(total_slots, jnp.float32) - jnp.float32(TOTAL))
    params = jnp.concatenate(
        [jnp.full((L,), jnp.asarray(log_scale, jnp.float32)),
         jnp.full((L,), zofs)])

    idx = idx.astype(jnp.int32)
    vals = unscaled_x.astype(jnp.float32)
    if n < W:
        pad = W - n
        idx_p = jnp.concatenate([idx, jnp.full((pad,), TOTAL, jnp.int32)])
        val_p = jnp.concatenate([vals, jnp.zeros((pad,), jnp.float32)])
        npad = W
    else:
        idx_p, val_p, npad = idx, vals, n

    t = max(0, n - L)
    tail_idx = lax.dynamic_slice_in_dim(idx_p, min(t, npad - L), L)
    tail_val = lax.dynamic_slice_in_dim(val_p, min(t, npad - L), L)

    return _sc_scatter(params, tail_idx, tail_val, idx_p, val_p, npad)
